# baseline (speedup 1.0000x reference)
import functools

import jax
import jax.numpy as jnp
from jax import lax
from jax.experimental import pallas as pl
from jax.experimental.pallas import tpu as pltpu

EPS = 1e-5
NC, NS = 2, 16
NW = NC * NS


def _sc_gather(idx_pad, emb_table, D):
    from jax.experimental.pallas import tpu_sc as plsc

    TPAD = idx_pad.shape[0]
    rows_per_w = TPAD // NW
    mesh = plsc.VectorSubcoreMesh(core_axis_name="c", subcore_axis_name="s")

    @functools.partial(
        pl.kernel,
        mesh=mesh,
        compiler_params=pltpu.CompilerParams(use_tc_tiling_on_sc=False),
        out_type=jax.ShapeDtypeStruct((TPAD, D), jnp.float32),
        scratch_types=[
            pltpu.VMEM((rows_per_w,), jnp.int32),
            pltpu.VMEM((rows_per_w, D), jnp.float32),
            pltpu.SemaphoreType.DMA,
        ],
    )
    def gather_k(idx_hbm, table_hbm, out_hbm, idx_v, rows_v, sem):
        wid = lax.axis_index("s") * NC + lax.axis_index("c")
        base = wid * rows_per_w
        pltpu.sync_copy(idx_hbm.at[pl.ds(base, rows_per_w)], idx_v)
        pltpu.async_copy(table_hbm.at[idx_v], rows_v, sem).wait()
        pltpu.sync_copy(rows_v, out_hbm.at[pl.ds(base, rows_per_w)])

    return gather_k(idx_pad, emb_table)


def _tc_body(C, L, D, TPAD,
             x_ref, rows_ref, mf_ref, gamma_ref, beta_ref, bias_ref, w_ref,
             out_ref, a_ref, v_ref):
    @pl.when(pl.program_id(0) == 0)
    def _init():
        rows = rows_ref[...]
        mu = jnp.mean(rows, axis=1, keepdims=True)
        xc = rows - mu
        var = jnp.mean(xc * xc, axis=1, keepdims=True)
        ln = xc * lax.rsqrt(var + EPS) * gamma_ref[...] + beta_ref[...]
        mf = mf_ref[...]
        lnm = ln * mf
        t_col = lax.broadcasted_iota(jnp.int32, (C, TPAD), 1) // L
        c_row = lax.broadcasted_iota(jnp.int32, (C, TPAD), 0)
        sel = jnp.where(t_col == c_row, 1.0, 0.0)
        pool = lax.dot(sel, lnm, preferred_element_type=jnp.float32)
        den = lax.dot(sel, mf, preferred_element_type=jnp.float32)
        col = pool / den
        a_mat = lax.dot_general(col, w_ref[...], (((1,), (1,)), ((), ())),
                                preferred_element_type=jnp.float32)
        v = lax.dot_general(bias_ref[...], w_ref[...], (((1,), (1,)), ((), ())),
                            preferred_element_type=jnp.float32)
        a_ref[...] = a_mat[None]
        v_ref[...] = v[None]

    x3 = lax.broadcast_in_dim(x_ref[...], (x_ref.shape[0], C, D), (0, 1))
    out_ref[...] = x3 * a_ref[...] + v_ref[...]


def kernel(x_num, num_col_input_ids, num_att_mask, emb_table, ln_gamma,
           ln_beta, num_bias, W_align):
    B, C = x_num.shape
    _, L = num_col_input_ids.shape
    D = emb_table.shape[1]
    T = C * L
    TPAD = ((T + 8 * NW - 1) // (8 * NW)) * (8 * NW)

    idx_pad = jnp.zeros((TPAD,), jnp.int32).at[:T].set(
        num_col_input_ids.reshape(-1))
    rows = _sc_gather(idx_pad, emb_table, D)

    mf_pad = jnp.zeros((TPAD, 1), jnp.float32).at[:T, :].set(
        num_att_mask.astype(jnp.float32).reshape(T, 1))

    BB = 128
    NBLK = B // BB
    out = pl.pallas_call(
        functools.partial(_tc_body, C, L, D, TPAD),
        grid=(NBLK,),
        in_specs=[
            pl.BlockSpec((BB, C), lambda i: (i, 0)),
            pl.BlockSpec((TPAD, D), lambda i: (0, 0)),
            pl.BlockSpec((TPAD, 1), lambda i: (0, 0)),
            pl.BlockSpec((1, D), lambda i: (0, 0)),
            pl.BlockSpec((1, D), lambda i: (0, 0)),
            pl.BlockSpec((1, D), lambda i: (0, 0)),
            pl.BlockSpec((D, D), lambda i: (0, 0)),
        ],
        out_specs=pl.BlockSpec((BB, C, D), lambda i: (i, 0, 0)),
        out_shape=jax.ShapeDtypeStruct((B, C, D), jnp.float32),
        scratch_shapes=[
            pltpu.VMEM((1, C, D), jnp.float32),
            pltpu.VMEM((1, 1, D), jnp.float32),
        ],
    )(x_num, rows, mf_pad, ln_gamma.reshape(1, D), ln_beta.reshape(1, D),
      num_bias.reshape(1, D), W_align)
    attention_mask = jnp.ones((B, C), dtype=jnp.float32)
    return out, attention_mask


# speedup vs baseline: 1.1414x; 1.1414x over previous
import functools

import jax
import jax.numpy as jnp
from jax import lax
from jax.experimental import pallas as pl
from jax.experimental.pallas import tpu as pltpu

EPS = 1e-5
NC, NS = 2, 16
NW = NC * NS


def _sc_gather(idx2, table2):
    from jax.experimental.pallas import tpu_sc as plsc

    TPAD = idx2.shape[0]
    D2 = table2.shape[1]
    rows_per_w = TPAD // NW
    mesh = plsc.VectorSubcoreMesh(core_axis_name="c", subcore_axis_name="s")

    @functools.partial(
        pl.kernel,
        mesh=mesh,
        out_type=jax.ShapeDtypeStruct((TPAD, D2), jnp.float32),
        scratch_types=[
            pltpu.VMEM((rows_per_w,), jnp.int32),
            pltpu.VMEM((rows_per_w, D2), jnp.float32),
            pltpu.SemaphoreType.DMA,
        ],
    )
    def gather_k(idx_hbm, table_hbm, out_hbm, idx_v, rows_v, sem):
        wid = lax.axis_index("s") * NC + lax.axis_index("c")
        base = wid * rows_per_w
        pltpu.sync_copy(idx_hbm.at[pl.ds(base, rows_per_w)], idx_v)
        pltpu.async_copy(table_hbm.at[idx_v], rows_v, sem).wait()
        pltpu.sync_copy(rows_v, out_hbm.at[pl.ds(base, rows_per_w)])

    return gather_k(idx2, table2)


def _prep_body(C, L, D, TPAD,
               rows2_ref, par_ref, mf_ref, gamma_ref, beta_ref, bias_ref,
               w_ref, a_ref, v_ref):
    rows2 = rows2_ref[...]
    rows = jnp.where(par_ref[...] == 0.0,
                     rows2[:, :D], rows2[:, D:])
    mu = jnp.mean(rows, axis=1, keepdims=True)
    xc = rows - mu
    var = jnp.mean(xc * xc, axis=1, keepdims=True)
    ln = xc * lax.rsqrt(var + EPS) * gamma_ref[...] + beta_ref[...]
    mf = mf_ref[...]
    lnm = ln * mf
    t_col = lax.broadcasted_iota(jnp.int32, (C, TPAD), 1) // L
    c_row = lax.broadcasted_iota(jnp.int32, (C, TPAD), 0)
    sel = jnp.where(t_col == c_row, 1.0, 0.0)
    pool = lax.dot(sel, lnm, preferred_element_type=jnp.float32)
    den = lax.dot(sel, mf, preferred_element_type=jnp.float32)
    col = pool / den
    a_ref[...] = lax.dot_general(col, w_ref[...], (((1,), (1,)), ((), ())),
                                 preferred_element_type=jnp.float32)
    v_ref[...] = lax.dot_general(bias_ref[...], w_ref[...],
                                 (((1,), (1,)), ((), ())),
                                 preferred_element_type=jnp.float32)


def _bcast_body(BB, D, xte_ref, xto_ref, a_ref, v_ref, out_ref):
    a = a_ref[...]
    v = v_ref[...]
    ae, ao = a[:, :D], a[:, D:]
    ve, vo = v[:, :D], v[:, D:]
    for b in range(BB):
        ce = xte_ref[:, b:b + 1]
        co = xto_ref[:, b:b + 1]
        out_ref[b] = jnp.concatenate(
            [ae * ce + ve, ao * co + vo], axis=1)


def kernel(x_num, num_col_input_ids, num_att_mask, emb_table, ln_gamma,
           ln_beta, num_bias, W_align):
    B, C = x_num.shape
    _, L = num_col_input_ids.shape
    V, D = emb_table.shape
    T = C * L
    TPAD = ((T + 8 * NW - 1) // (8 * NW)) * (8 * NW)

    idx_pad = jnp.zeros((TPAD,), jnp.int32).at[:T].set(
        num_col_input_ids.reshape(-1))
    table2 = emb_table.reshape(V // 2, 2 * D)
    rows2 = _sc_gather(idx_pad // 2, table2)
    par = (idx_pad % 2).astype(jnp.float32).reshape(TPAD, 1)

    mf_pad = jnp.zeros((TPAD, 1), jnp.float32).at[:T, :].set(
        num_att_mask.astype(jnp.float32).reshape(T, 1))

    a_mat, v_vec = pl.pallas_call(
        functools.partial(_prep_body, C, L, D, TPAD),
        out_shape=[jax.ShapeDtypeStruct((C, D), jnp.float32),
                   jax.ShapeDtypeStruct((1, D), jnp.float32)],
    )(rows2, par, mf_pad, ln_gamma.reshape(1, D), ln_beta.reshape(1, D),
      num_bias.reshape(1, D), W_align)

    a128 = a_mat.reshape(C // 2, 2 * D)
    v128 = jnp.concatenate([v_vec, v_vec], axis=1)
    xte = x_num[:, 0::2].T
    xto = x_num[:, 1::2].T

    BB = 128
    out128 = pl.pallas_call(
        functools.partial(_bcast_body, BB, D),
        grid=(B // BB,),
        in_specs=[
            pl.BlockSpec((C // 2, BB), lambda i: (0, i)),
            pl.BlockSpec((C // 2, BB), lambda i: (0, i)),
            pl.BlockSpec((C // 2, 2 * D), lambda i: (0, 0)),
            pl.BlockSpec((1, 2 * D), lambda i: (0, 0)),
        ],
        out_specs=pl.BlockSpec((BB, C // 2, 2 * D), lambda i: (i, 0, 0)),
        out_shape=jax.ShapeDtypeStruct((B, C // 2, 2 * D), jnp.float32),
    )(xte, xto, a128, v128)

    out = out128.reshape(B, C, D)
    attention_mask = jnp.ones((B, C), dtype=jnp.float32)
    return out, attention_mask
